# baseline (speedup 1.0000x reference)
import functools

import jax
import jax.numpy as jnp
from jax import lax
from jax.experimental import pallas as pl
from jax.experimental.pallas import tpu as pltpu
from jax.experimental.pallas import tpu_sc as plsc

N_LOCS = 100000
N_CLUSTERS = 10000
E = 320000
D = 128

NC = 2
NS = 16
NW = NC * NS
CPT = 320
LAST = N_CLUSTERS - (NW - 1) * CPT
CHUNK = 8000
NCHUNKS = E // CHUNK
GROUPS = CHUNK // 16
SG = 64
HCAP = CHUNK + SG
NEG = float("-inf")

_mesh = plsc.VectorSubcoreMesh(core_axis_name="c", subcore_axis_name="s")


@functools.partial(
    pl.kernel,
    out_type=jax.ShapeDtypeStruct((N_CLUSTERS, 2 * D), jnp.float32),
    mesh=_mesh,
    scratch_types=[
        pltpu.VMEM((CPT + 1, D), jnp.float32),
        pltpu.VMEM((2 * CHUNK,), jnp.int32),
        pltpu.VMEM((2 * CHUNK,), jnp.int32),
        pltpu.VMEM((HCAP,), jnp.int32),
        pltpu.VMEM((HCAP,), jnp.int32),
        pltpu.VMEM((2 * SG, D), jnp.float32),
        pltpu.SemaphoreType.DMA((2,)),
        pltpu.SemaphoreType.DMA((2,)),
        pltpu.SemaphoreType.DMA((2,)),
    ],
    compiler_params=pltpu.CompilerParams(needs_layout_passes=False),
)
def _loc2cluster(x_locs, x_clusters, src_h, dst_h, out,
                 acc, dstbuf, srcbuf, hitdst, hitsrc, rowfl,
                 semd, sems, semg):
    wid = lax.axis_index("s") * NC + lax.axis_index("c")
    lo = wid * CPT
    hi = lo + jnp.where(wid == NW - 1, LAST, CPT)
    lanes = lax.iota(jnp.int32, 16)
    neg16 = jnp.full((16,), NEG, jnp.float32)

    def init_row(r, _):
        for kk in range(D // 16):
            acc[r, pl.ds(kk * 16, 16)] = neg16
        return 0

    lax.fori_loop(0, CPT + 1, init_row, 0)

    def chunk_copies(c):
        par = lax.rem(c, 2)
        base = par * CHUNK
        cd = pltpu.make_async_copy(dst_h.at[pl.ds(c * CHUNK, CHUNK)],
                                   dstbuf.at[pl.ds(base, CHUNK)], semd.at[par])
        cs = pltpu.make_async_copy(src_h.at[pl.ds(c * CHUNK, CHUNK)],
                                   srcbuf.at[pl.ds(base, CHUNK)], sems.at[par])
        return cd, cs

    def start_chunk(c):
        cd, cs = chunk_copies(c)
        cd.start()
        cs.start()

    def wait_chunk(c):
        cd, cs = chunk_copies(c)
        cd.wait()
        cs.wait()

    def gather_copy(gbase, par):
        return pltpu.make_async_copy(x_locs.at[hitsrc.at[pl.ds(gbase, SG)]],
                                     rowfl.at[pl.ds(par * SG, SG)],
                                     semg.at[par])

    def update_from(gbase, par):
        def upd16(q, _):
            dsts16 = hitdst[pl.ds(gbase + q * 16, 16)]
            rbase = par * SG + q * 16
            for j in range(16):
                drow = dsts16[j]
                for kk in range(D // 16):
                    sl = pl.ds(kk * 16, 16)
                    acc[drow, sl] = jnp.maximum(acc[drow, sl],
                                                rowfl[rbase + j, sl])
            return 0

        lax.fori_loop(0, SG // 16, upd16, 0)

    start_chunk(0)

    def chunk_body(c, hcnt):
        @pl.when(c + 1 < NCHUNKS)
        def _():
            start_chunk(c + 1)

        wait_chunk(c)
        base = lax.rem(c, 2) * CHUNK

        def group_body(g, hv):
            for u in range(2):
                off = base + g * 32 + u * 16
                d16 = dstbuf[pl.ds(off, 16)]
                s16 = srcbuf[pl.ds(off, 16)]
                m = (d16 >= lo) & (d16 < hi)
                cntv = plsc.all_reduce_population_count(m)
                pos = hv + plsc.cumsum(jnp.where(m, 1, 0)) - 1
                plsc.store_scatter(hitdst, [pos], d16 - lo, mask=m)
                plsc.store_scatter(hitsrc, [pos], s16, mask=m)
                hv = hv + cntv
            return hv

        hv0 = jnp.full((16,), hcnt, jnp.int32)
        hcnt = lax.fori_loop(0, GROUPS // 2, group_body, hv0)[0]

        ng = hcnt // SG

        @pl.when(ng > 0)
        def _():
            gather_copy(0, 0).start()

        def drain(g, _):
            par = lax.rem(g, 2)

            @pl.when(g + 1 < ng)
            def _():
                gather_copy((g + 1) * SG, 1 - par).start()

            gather_copy(g * SG, par).wait()
            update_from(g * SG, par)
            return 0

        lax.fori_loop(0, ng, drain, 0)

        rem = hcnt - ng * SG
        for q in range(SG // 16):
            d16 = hitdst[pl.ds(ng * SG + q * 16, 16)]
            s16 = hitsrc[pl.ds(ng * SG + q * 16, 16)]
            hitdst[pl.ds(q * 16, 16)] = d16
            hitsrc[pl.ds(q * 16, 16)] = s16
        return rem

    rem = lax.fori_loop(0, NCHUNKS, chunk_body, jnp.int32(0))

    @pl.when(rem > 0)
    def _():
        for q in range(SG // 16):
            d16 = hitdst[pl.ds(q * 16, 16)]
            s16 = hitsrc[pl.ds(q * 16, 16)]
            msk = (lanes + q * 16) < rem
            hitdst[pl.ds(q * 16, 16)] = jnp.where(msk, d16, CPT)
            hitsrc[pl.ds(q * 16, 16)] = jnp.where(msk, s16, 0)
        gc = gather_copy(0, 0)
        gc.start()
        gc.wait()
        update_from(0, 0)

    def fix_row(r, _):
        for kk in range(D // 16):
            sl = pl.ds(kk * 16, 16)
            v = acc[r, sl]
            acc[r, sl] = jnp.where(v == NEG, 0.0, v)
        return 0

    lax.fori_loop(0, CPT, fix_row, 0)

    def copy_clusters(row0, n):
        pltpu.sync_copy(x_clusters.at[pl.ds(lo + row0, n)],
                        rowfl.at[pl.ds(0, n)])
        pltpu.sync_copy(rowfl.at[pl.ds(0, n)],
                        out.at[pl.ds(lo + row0, n), pl.ds(0, D)])

    @pl.when(wid < NW - 1)
    def _():
        copy_clusters(0, 128)
        copy_clusters(128, 128)
        copy_clusters(256, 64)
        pltpu.sync_copy(acc.at[pl.ds(0, CPT)], out.at[pl.ds(lo, CPT), pl.ds(D, D)])

    @pl.when(wid == NW - 1)
    def _():
        copy_clusters(0, LAST)
        pltpu.sync_copy(acc.at[pl.ds(0, LAST)], out.at[pl.ds(lo, LAST), pl.ds(D, D)])


def kernel(x_locs, x_clusters, edge_src, edge_dst):
    edge_src = edge_src.astype(jnp.int32)
    edge_dst = edge_dst.astype(jnp.int32)
    return _loc2cluster(x_locs, x_clusters, edge_src, edge_dst)


# speedup vs baseline: 1.1377x; 1.1377x over previous
import functools

import jax
import jax.numpy as jnp
from jax import lax
from jax.experimental import pallas as pl
from jax.experimental.pallas import tpu as pltpu
from jax.experimental.pallas import tpu_sc as plsc

N_LOCS = 100000
N_CLUSTERS = 10000
E = 320000
D = 128

NC = 2
NS = 16
NW = NC * NS
CPT = 320
LAST = N_CLUSTERS - (NW - 1) * CPT
CHUNK = 8000
HALF = CHUNK // 2
NCHUNKS = E // CHUNK
GROUPS2 = HALF // 16
SG = 64
HCAPS = HALF + SG
NEG = float("-inf")

_mesh = plsc.VectorSubcoreMesh(core_axis_name="c", subcore_axis_name="s")


@functools.partial(
    pl.kernel,
    out_type=jax.ShapeDtypeStruct((N_CLUSTERS, 2 * D), jnp.float32),
    mesh=_mesh,
    scratch_types=[
        pltpu.VMEM((CPT + 1, D), jnp.float32),
        pltpu.VMEM((2 * CHUNK,), jnp.int32),
        pltpu.VMEM((2 * CHUNK,), jnp.int32),
        pltpu.VMEM((2 * HCAPS,), jnp.int32),
        pltpu.VMEM((2 * HCAPS,), jnp.int32),
        pltpu.VMEM((2 * SG, D), jnp.float32),
        pltpu.SemaphoreType.DMA((2,)),
        pltpu.SemaphoreType.DMA((2,)),
        pltpu.SemaphoreType.DMA((2,)),
    ],
    compiler_params=pltpu.CompilerParams(needs_layout_passes=False),
)
def _loc2cluster(x_locs, x_clusters, src_h, dst_h, out,
                 acc, dstbuf, srcbuf, hitdst, hitsrc, rowfl,
                 semd, sems, semg):
    wid = lax.axis_index("s") * NC + lax.axis_index("c")
    lo = wid * CPT
    hi = lo + jnp.where(wid == NW - 1, LAST, CPT)
    lanes = lax.iota(jnp.int32, 16)
    neg16 = jnp.full((16,), NEG, jnp.float32)

    def init_row(r, _):
        for kk in range(D // 16):
            acc[r, pl.ds(kk * 16, 16)] = neg16
        return 0

    lax.fori_loop(0, CPT + 1, init_row, 0)

    def chunk_copies(c):
        par = lax.rem(c, 2)
        base = par * CHUNK
        cd = pltpu.make_async_copy(dst_h.at[pl.ds(c * CHUNK, CHUNK)],
                                   dstbuf.at[pl.ds(base, CHUNK)], semd.at[par])
        cs = pltpu.make_async_copy(src_h.at[pl.ds(c * CHUNK, CHUNK)],
                                   srcbuf.at[pl.ds(base, CHUNK)], sems.at[par])
        return cd, cs

    def start_chunk(c):
        cd, cs = chunk_copies(c)
        cd.start()
        cs.start()

    def wait_chunk(c):
        cd, cs = chunk_copies(c)
        cd.wait()
        cs.wait()

    def gather_copy(gbase, par):
        return pltpu.make_async_copy(x_locs.at[hitsrc.at[pl.ds(gbase, SG)]],
                                     rowfl.at[pl.ds(par * SG, SG)],
                                     semg.at[par])

    def update_from(gbase, par):
        def upd16(q, _):
            dsts16 = hitdst[pl.ds(gbase + q * 16, 16)]
            rbase = par * SG + q * 16
            for j in range(16):
                drow = dsts16[j]
                for kk in range(D // 16):
                    sl = pl.ds(kk * 16, 16)
                    acc[drow, sl] = jnp.maximum(acc[drow, sl],
                                                rowfl[rbase + j, sl])
            return 0

        lax.fori_loop(0, SG // 16, upd16, 0)

    def drain_stream(sbase, hc):
        ng = hc // SG

        @pl.when(ng > 0)
        def _():
            gather_copy(sbase, 0).start()

        def drain(g, _):
            par = lax.rem(g, 2)

            @pl.when(g + 1 < ng)
            def _():
                gather_copy(sbase + (g + 1) * SG, 1 - par).start()

            gather_copy(sbase + g * SG, par).wait()
            update_from(sbase + g * SG, par)
            return 0

        lax.fori_loop(0, ng, drain, 0)

        rem = hc - ng * SG
        for q in range(SG // 16):
            d16 = hitdst[pl.ds(sbase + ng * SG + q * 16, 16)]
            s16 = hitsrc[pl.ds(sbase + ng * SG + q * 16, 16)]
            hitdst[pl.ds(sbase + q * 16, 16)] = d16
            hitsrc[pl.ds(sbase + q * 16, 16)] = s16
        return rem

    start_chunk(0)

    def chunk_body(c, carry):
        hcA, hcB = carry

        @pl.when(c + 1 < NCHUNKS)
        def _():
            start_chunk(c + 1)

        wait_chunk(c)
        base = lax.rem(c, 2) * CHUNK

        def group_body(g, hh):
            hA, hB = hh
            offA = base + g * 16
            offB = offA + HALF
            dA = dstbuf[pl.ds(offA, 16)]
            sA = srcbuf[pl.ds(offA, 16)]
            dB = dstbuf[pl.ds(offB, 16)]
            sB = srcbuf[pl.ds(offB, 16)]
            mA = (dA >= lo) & (dA < hi)
            mB = (dB >= lo) & (dB < hi)
            cA = plsc.all_reduce_population_count(mA)[0]
            cB = plsc.all_reduce_population_count(mB)[0]
            plsc.store_compressed(hitdst.at[pl.ds(hA, 16)], dA - lo, mask=mA)
            plsc.store_compressed(hitsrc.at[pl.ds(hA, 16)], sA, mask=mA)
            plsc.store_compressed(hitdst.at[pl.ds(HCAPS + hB, 16)], dB - lo,
                                  mask=mB)
            plsc.store_compressed(hitsrc.at[pl.ds(HCAPS + hB, 16)], sB,
                                  mask=mB)
            return hA + cA, hB + cB

        hcA, hcB = lax.fori_loop(0, GROUPS2, group_body, (hcA, hcB))
        hcA = drain_stream(0, hcA)
        hcB = drain_stream(HCAPS, hcB)
        return hcA, hcB

    remA, remB = lax.fori_loop(0, NCHUNKS, chunk_body,
                               (jnp.int32(0), jnp.int32(0)))

    def final_flush(sbase, rem):
        @pl.when(rem > 0)
        def _():
            for q in range(SG // 16):
                d16 = hitdst[pl.ds(sbase + q * 16, 16)]
                s16 = hitsrc[pl.ds(sbase + q * 16, 16)]
                msk = (lanes + q * 16) < rem
                hitdst[pl.ds(sbase + q * 16, 16)] = jnp.where(msk, d16, CPT)
                hitsrc[pl.ds(sbase + q * 16, 16)] = jnp.where(msk, s16, 0)
            gc = gather_copy(sbase, 0)
            gc.start()
            gc.wait()
            update_from(sbase, 0)

    final_flush(0, remA)
    final_flush(HCAPS, remB)

    def fix_row(r, _):
        for kk in range(D // 16):
            sl = pl.ds(kk * 16, 16)
            v = acc[r, sl]
            acc[r, sl] = jnp.where(v == NEG, 0.0, v)
        return 0

    lax.fori_loop(0, CPT, fix_row, 0)

    def copy_clusters(row0, n):
        pltpu.sync_copy(x_clusters.at[pl.ds(lo + row0, n)],
                        rowfl.at[pl.ds(0, n)])
        pltpu.sync_copy(rowfl.at[pl.ds(0, n)],
                        out.at[pl.ds(lo + row0, n), pl.ds(0, D)])

    @pl.when(wid < NW - 1)
    def _():
        copy_clusters(0, 128)
        copy_clusters(128, 128)
        copy_clusters(256, 64)
        pltpu.sync_copy(acc.at[pl.ds(0, CPT)], out.at[pl.ds(lo, CPT), pl.ds(D, D)])

    @pl.when(wid == NW - 1)
    def _():
        copy_clusters(0, LAST)
        pltpu.sync_copy(acc.at[pl.ds(0, LAST)], out.at[pl.ds(lo, LAST), pl.ds(D, D)])


def kernel(x_locs, x_clusters, edge_src, edge_dst):
    edge_src = edge_src.astype(jnp.int32)
    edge_dst = edge_dst.astype(jnp.int32)
    return _loc2cluster(x_locs, x_clusters, edge_src, edge_dst)


# speedup vs baseline: 1.3024x; 1.1447x over previous
import functools

import jax
import jax.numpy as jnp
from jax import lax
from jax.experimental import pallas as pl
from jax.experimental.pallas import tpu as pltpu
from jax.experimental.pallas import tpu_sc as plsc

N_LOCS = 100000
N_CLUSTERS = 10000
E = 320000
D = 128

NC = 2
NS = 16
NW = NC * NS
CPT = 320
LAST = N_CLUSTERS - (NW - 1) * CPT
CHUNK = 8000
HALF = CHUNK // 2
NCHUNKS = E // CHUNK
GROUPS2 = HALF // 16
SG = 64
HCAPS = HALF + SG
NEG = float("-inf")

_mesh = plsc.VectorSubcoreMesh(core_axis_name="c", subcore_axis_name="s")


@functools.partial(
    pl.kernel,
    out_type=jax.ShapeDtypeStruct((N_CLUSTERS, 2 * D), jnp.float32),
    mesh=_mesh,
    scratch_types=[
        pltpu.VMEM((CPT + 1, D), jnp.float32),
        pltpu.VMEM((2 * CHUNK,), jnp.int32),
        pltpu.VMEM((2 * CHUNK,), jnp.int32),
        pltpu.VMEM((2 * HCAPS,), jnp.int32),
        pltpu.VMEM((2 * HCAPS,), jnp.int32),
        pltpu.VMEM((2 * SG, D), jnp.float32),
        pltpu.VMEM((2 * SG, D), jnp.float32),
        pltpu.SemaphoreType.DMA((2,)),
        pltpu.SemaphoreType.DMA((2,)),
        pltpu.SemaphoreType.DMA((2,)),
        pltpu.SemaphoreType.DMA((2,)),
    ],
    compiler_params=pltpu.CompilerParams(needs_layout_passes=False),
)
def _loc2cluster(x_locs, x_clusters, src_h, dst_h, out,
                 acc, dstbuf, srcbuf, hitdst, hitsrc, rowflA, rowflB,
                 semd, sems, semgA, semgB):
    wid = lax.axis_index("s") * NC + lax.axis_index("c")
    lo = wid * CPT
    hi = lo + jnp.where(wid == NW - 1, LAST, CPT)
    lanes = lax.iota(jnp.int32, 16)
    neg16 = jnp.full((16,), NEG, jnp.float32)

    def init_row(r, _):
        for kk in range(D // 16):
            acc[r, pl.ds(kk * 16, 16)] = neg16
        return 0

    lax.fori_loop(0, CPT + 1, init_row, 0)

    def chunk_copies(c):
        par = lax.rem(c, 2)
        base = par * CHUNK
        cd = pltpu.make_async_copy(dst_h.at[pl.ds(c * CHUNK, CHUNK)],
                                   dstbuf.at[pl.ds(base, CHUNK)], semd.at[par])
        cs = pltpu.make_async_copy(src_h.at[pl.ds(c * CHUNK, CHUNK)],
                                   srcbuf.at[pl.ds(base, CHUNK)], sems.at[par])
        return cd, cs

    def start_chunk(c):
        cd, cs = chunk_copies(c)
        cd.start()
        cs.start()

    def wait_chunk(c):
        cd, cs = chunk_copies(c)
        cd.wait()
        cs.wait()

    def gather_copy(rf, sem, gbase, par):
        return pltpu.make_async_copy(x_locs.at[hitsrc.at[pl.ds(gbase, SG)]],
                                     rf.at[pl.ds(par * SG, SG)],
                                     sem.at[par])

    def update_from(rf, gbase, par):
        def upd16(q, _):
            dsts16 = hitdst[pl.ds(gbase + q * 16, 16)]
            rbase = par * SG + q * 16
            for j in range(16):
                drow = dsts16[j]
                for kk in range(D // 16):
                    sl = pl.ds(kk * 16, 16)
                    acc[drow, sl] = jnp.maximum(acc[drow, sl],
                                                rf[rbase + j, sl])
            return 0

        lax.fori_loop(0, SG // 16, upd16, 0)

    def drain_stream(rf, sem, sbase, hc, pre):
        ng = hc // SG

        @pl.when((ng > 0) & jnp.logical_not(pre))
        def _():
            gather_copy(rf, sem, sbase, 0).start()

        def drain(g, _):
            par = lax.rem(g, 2)

            @pl.when(g + 1 < ng)
            def _():
                gather_copy(rf, sem, sbase + (g + 1) * SG, 1 - par).start()

            gather_copy(rf, sem, sbase + g * SG, par).wait()
            update_from(rf, sbase + g * SG, par)
            return 0

        lax.fori_loop(0, ng, drain, 0)

        rem = hc - ng * SG
        for q in range(SG // 16):
            d16 = hitdst[pl.ds(sbase + ng * SG + q * 16, 16)]
            s16 = hitsrc[pl.ds(sbase + ng * SG + q * 16, 16)]
            hitdst[pl.ds(sbase + q * 16, 16)] = d16
            hitsrc[pl.ds(sbase + q * 16, 16)] = s16
        return rem

    start_chunk(0)

    def chunk_body(c, carry):
        hcA, hcB = carry

        @pl.when(c + 1 < NCHUNKS)
        def _():
            start_chunk(c + 1)

        wait_chunk(c)
        base = lax.rem(c, 2) * CHUNK

        def group_body(g, hh):
            hA, hB = hh
            offA = base + g * 16
            offB = offA + HALF
            dA = dstbuf[pl.ds(offA, 16)]
            sA = srcbuf[pl.ds(offA, 16)]
            dB = dstbuf[pl.ds(offB, 16)]
            sB = srcbuf[pl.ds(offB, 16)]
            mA = (dA >= lo) & (dA < hi)
            mB = (dB >= lo) & (dB < hi)
            cA = plsc.all_reduce_population_count(mA)[0]
            cB = plsc.all_reduce_population_count(mB)[0]
            plsc.store_compressed(hitdst.at[pl.ds(hA, 16)], dA - lo, mask=mA)
            plsc.store_compressed(hitsrc.at[pl.ds(hA, 16)], sA, mask=mA)
            plsc.store_compressed(hitdst.at[pl.ds(HCAPS + hB, 16)], dB - lo,
                                  mask=mB)
            plsc.store_compressed(hitsrc.at[pl.ds(HCAPS + hB, 16)], sB,
                                  mask=mB)
            return hA + cA, hB + cB

        half = GROUPS2 // 2
        hmA, hmB = lax.fori_loop(0, half, group_body, (hcA, hcB))

        preA = hmA >= SG
        preB = hmB >= SG

        @pl.when(preA)
        def _():
            gather_copy(rowflA, semgA, 0, 0).start()

        @pl.when(preB)
        def _():
            gather_copy(rowflB, semgB, HCAPS, 0).start()

        hcA, hcB = lax.fori_loop(half, GROUPS2, group_body, (hmA, hmB))
        hcA = drain_stream(rowflA, semgA, 0, hcA, preA)
        hcB = drain_stream(rowflB, semgB, HCAPS, hcB, preB)
        return hcA, hcB

    remA, remB = lax.fori_loop(0, NCHUNKS, chunk_body,
                               (jnp.int32(0), jnp.int32(0)))

    def final_flush(rf, sem, sbase, rem):
        @pl.when(rem > 0)
        def _():
            for q in range(SG // 16):
                d16 = hitdst[pl.ds(sbase + q * 16, 16)]
                s16 = hitsrc[pl.ds(sbase + q * 16, 16)]
                msk = (lanes + q * 16) < rem
                hitdst[pl.ds(sbase + q * 16, 16)] = jnp.where(msk, d16, CPT)
                hitsrc[pl.ds(sbase + q * 16, 16)] = jnp.where(msk, s16, 0)
            gc = gather_copy(rf, sem, sbase, 0)
            gc.start()
            gc.wait()
            update_from(rf, sbase, 0)

    final_flush(rowflA, semgA, 0, remA)
    final_flush(rowflB, semgB, HCAPS, remB)

    def fix_row(r, _):
        for kk in range(D // 16):
            sl = pl.ds(kk * 16, 16)
            v = acc[r, sl]
            acc[r, sl] = jnp.where(v == NEG, 0.0, v)
        return 0

    lax.fori_loop(0, CPT, fix_row, 0)

    def copy_clusters(row0, n):
        pltpu.sync_copy(x_clusters.at[pl.ds(lo + row0, n)],
                        rowflA.at[pl.ds(0, n)])
        pltpu.sync_copy(rowflA.at[pl.ds(0, n)],
                        out.at[pl.ds(lo + row0, n), pl.ds(0, D)])

    @pl.when(wid < NW - 1)
    def _():
        copy_clusters(0, 128)
        copy_clusters(128, 128)
        copy_clusters(256, 64)
        pltpu.sync_copy(acc.at[pl.ds(0, CPT)], out.at[pl.ds(lo, CPT), pl.ds(D, D)])

    @pl.when(wid == NW - 1)
    def _():
        copy_clusters(0, LAST)
        pltpu.sync_copy(acc.at[pl.ds(0, LAST)], out.at[pl.ds(lo, LAST), pl.ds(D, D)])


def kernel(x_locs, x_clusters, edge_src, edge_dst):
    edge_src = edge_src.astype(jnp.int32)
    edge_dst = edge_dst.astype(jnp.int32)
    return _loc2cluster(x_locs, x_clusters, edge_src, edge_dst)


# speedup vs baseline: 1.3209x; 1.0142x over previous
import functools

import jax
import jax.numpy as jnp
from jax import lax
from jax.experimental import pallas as pl
from jax.experimental.pallas import tpu as pltpu
from jax.experimental.pallas import tpu_sc as plsc

N_LOCS = 100000
N_CLUSTERS = 10000
E = 320000
D = 128

NC = 2
NS = 16
NW = NC * NS
CPT = 320
LAST = N_CLUSTERS - (NW - 1) * CPT
CHUNK = 8000
HALF = CHUNK // 2
NCHUNKS = E // CHUNK
GROUPS2 = HALF // 16
SG = 64
HCAPS = HALF + SG
NEG = float("-inf")

_mesh = plsc.VectorSubcoreMesh(core_axis_name="c", subcore_axis_name="s")


@functools.partial(
    pl.kernel,
    out_type=jax.ShapeDtypeStruct((N_CLUSTERS, 2 * D), jnp.float32),
    mesh=_mesh,
    scratch_types=[
        pltpu.VMEM((CPT + 1, D), jnp.float32),
        pltpu.VMEM((2 * CHUNK,), jnp.int32),
        pltpu.VMEM((2 * CHUNK,), jnp.int32),
        pltpu.VMEM((2 * HCAPS,), jnp.int32),
        pltpu.VMEM((2 * HCAPS,), jnp.int32),
        pltpu.VMEM((2 * SG, D), jnp.float32),
        pltpu.VMEM((2 * SG, D), jnp.float32),
        pltpu.SemaphoreType.DMA((2,)),
        pltpu.SemaphoreType.DMA((2,)),
        pltpu.SemaphoreType.DMA((2,)),
        pltpu.SemaphoreType.DMA((2,)),
        pltpu.SemaphoreType.DMA,
    ],
    compiler_params=pltpu.CompilerParams(needs_layout_passes=False),
)
def _loc2cluster(x_locs, x_clusters, src_h, dst_h, out,
                 acc, dstbuf, srcbuf, hitdst, hitsrc, rowflA, rowflB,
                 semd, sems, semgA, semgB, semc):
    wid = lax.axis_index("s") * NC + lax.axis_index("c")
    lo = wid * CPT
    hi = lo + jnp.where(wid == NW - 1, LAST, CPT)
    lanes = lax.iota(jnp.int32, 16)
    neg16 = jnp.full((16,), NEG, jnp.float32)

    def cl_copy(n):
        return pltpu.make_async_copy(x_clusters.at[pl.ds(lo, n)],
                                     out.at[pl.ds(lo, n), pl.ds(0, D)], semc)

    @pl.when(wid < NW - 1)
    def _():
        cl_copy(CPT).start()

    @pl.when(wid == NW - 1)
    def _():
        cl_copy(LAST).start()

    def init_row(r, _):
        for kk in range(D // 16):
            acc[r, pl.ds(kk * 16, 16)] = neg16
        return 0

    lax.fori_loop(0, CPT + 1, init_row, 0)

    def chunk_copies(c):
        par = lax.rem(c, 2)
        base = par * CHUNK
        cd = pltpu.make_async_copy(dst_h.at[pl.ds(c * CHUNK, CHUNK)],
                                   dstbuf.at[pl.ds(base, CHUNK)], semd.at[par])
        cs = pltpu.make_async_copy(src_h.at[pl.ds(c * CHUNK, CHUNK)],
                                   srcbuf.at[pl.ds(base, CHUNK)], sems.at[par])
        return cd, cs

    def start_chunk(c):
        cd, cs = chunk_copies(c)
        cd.start()
        cs.start()

    def wait_chunk(c):
        cd, cs = chunk_copies(c)
        cd.wait()
        cs.wait()

    def gather_copy(rf, sem, gbase, par):
        return pltpu.make_async_copy(x_locs.at[hitsrc.at[pl.ds(gbase, SG)]],
                                     rf.at[pl.ds(par * SG, SG)],
                                     sem.at[par])

    def update_from(rf, gbase, par):
        def upd16(q, _):
            dsts16 = hitdst[pl.ds(gbase + q * 16, 16)]
            rbase = par * SG + q * 16
            for j in range(16):
                drow = dsts16[j]
                for kk in range(D // 16):
                    sl = pl.ds(kk * 16, 16)
                    acc[drow, sl] = jnp.maximum(acc[drow, sl],
                                                rf[rbase + j, sl])
            return 0

        lax.fori_loop(0, SG // 16, upd16, 0)

    def drain_stream(rf, sem, sbase, hc, pre):
        ng = hc // SG

        @pl.when((ng > 0) & jnp.logical_not(pre))
        def _():
            gather_copy(rf, sem, sbase, 0).start()

        def drain(g, _):
            par = lax.rem(g, 2)

            @pl.when(g + 1 < ng)
            def _():
                gather_copy(rf, sem, sbase + (g + 1) * SG, 1 - par).start()

            gather_copy(rf, sem, sbase + g * SG, par).wait()
            update_from(rf, sbase + g * SG, par)
            return 0

        lax.fori_loop(0, ng, drain, 0)

        rem = hc - ng * SG
        for q in range(SG // 16):
            d16 = hitdst[pl.ds(sbase + ng * SG + q * 16, 16)]
            s16 = hitsrc[pl.ds(sbase + ng * SG + q * 16, 16)]
            hitdst[pl.ds(sbase + q * 16, 16)] = d16
            hitsrc[pl.ds(sbase + q * 16, 16)] = s16
        return rem

    start_chunk(0)

    def chunk_body(c, carry):
        hcA, hcB = carry

        @pl.when(c + 1 < NCHUNKS)
        def _():
            start_chunk(c + 1)

        wait_chunk(c)
        base = lax.rem(c, 2) * CHUNK

        def group_body(g, hh):
            hA, hB = hh
            offA = base + g * 16
            offB = offA + HALF
            dA = dstbuf[pl.ds(offA, 16)]
            sA = srcbuf[pl.ds(offA, 16)]
            dB = dstbuf[pl.ds(offB, 16)]
            sB = srcbuf[pl.ds(offB, 16)]
            mA = (dA >= lo) & (dA < hi)
            mB = (dB >= lo) & (dB < hi)
            cA = plsc.all_reduce_population_count(mA)[0]
            cB = plsc.all_reduce_population_count(mB)[0]
            plsc.store_compressed(hitdst.at[pl.ds(hA, 16)], dA - lo, mask=mA)
            plsc.store_compressed(hitsrc.at[pl.ds(hA, 16)], sA, mask=mA)
            plsc.store_compressed(hitdst.at[pl.ds(HCAPS + hB, 16)], dB - lo,
                                  mask=mB)
            plsc.store_compressed(hitsrc.at[pl.ds(HCAPS + hB, 16)], sB,
                                  mask=mB)
            return hA + cA, hB + cB

        half = GROUPS2 // 2
        hmA, hmB = lax.fori_loop(0, half, group_body, (hcA, hcB))

        preA = hmA >= SG
        preB = hmB >= SG

        @pl.when(preA)
        def _():
            gather_copy(rowflA, semgA, 0, 0).start()

        @pl.when(preB)
        def _():
            gather_copy(rowflB, semgB, HCAPS, 0).start()

        hcA, hcB = lax.fori_loop(half, GROUPS2, group_body, (hmA, hmB))
        hcA = drain_stream(rowflA, semgA, 0, hcA, preA)
        hcB = drain_stream(rowflB, semgB, HCAPS, hcB, preB)
        return hcA, hcB

    remA, remB = lax.fori_loop(0, NCHUNKS, chunk_body,
                               (jnp.int32(0), jnp.int32(0)))

    def final_flush(rf, sem, sbase, rem):
        @pl.when(rem > 0)
        def _():
            for q in range(SG // 16):
                d16 = hitdst[pl.ds(sbase + q * 16, 16)]
                s16 = hitsrc[pl.ds(sbase + q * 16, 16)]
                msk = (lanes + q * 16) < rem
                hitdst[pl.ds(sbase + q * 16, 16)] = jnp.where(msk, d16, CPT)
                hitsrc[pl.ds(sbase + q * 16, 16)] = jnp.where(msk, s16, 0)
            gc = gather_copy(rf, sem, sbase, 0)
            gc.start()
            gc.wait()
            update_from(rf, sbase, 0)

    final_flush(rowflA, semgA, 0, remA)
    final_flush(rowflB, semgB, HCAPS, remB)

    def fix_row(r, _):
        for kk in range(D // 16):
            sl = pl.ds(kk * 16, 16)
            v = acc[r, sl]
            acc[r, sl] = jnp.where(v == NEG, 0.0, v)
        return 0

    lax.fori_loop(0, CPT, fix_row, 0)

    @pl.when(wid < NW - 1)
    def _():
        pltpu.sync_copy(acc.at[pl.ds(0, CPT)], out.at[pl.ds(lo, CPT), pl.ds(D, D)])
        cl_copy(CPT).wait()

    @pl.when(wid == NW - 1)
    def _():
        pltpu.sync_copy(acc.at[pl.ds(0, LAST)], out.at[pl.ds(lo, LAST), pl.ds(D, D)])
        cl_copy(LAST).wait()


def kernel(x_locs, x_clusters, edge_src, edge_dst):
    edge_src = edge_src.astype(jnp.int32)
    edge_dst = edge_dst.astype(jnp.int32)
    return _loc2cluster(x_locs, x_clusters, edge_src, edge_dst)
